# baseline (speedup 1.0000x reference)
import functools

import jax
import jax.numpy as jnp
from jax.experimental import pallas as pl
from jax.experimental.pallas import tpu as pltpu


def _fused_kernel(
    x_ref,
    adj_ref,
    wfc_ref,
    bfc_ref,
    wg1_ref,
    bg1_ref,
    wg2_ref,
    bg2_ref,
    out_ref,
    h1_ref,
    h2_ref,
    *,
    n,
    nblk,
    bm,
):
    i = pl.program_id(0)

    @pl.when(i == 0)
    def _():
        h = jnp.dot(x_ref[...], wfc_ref[...], preferred_element_type=jnp.float32)
        h = jnp.maximum(h + bfc_ref[...], 0.0)
        h1_ref[...] = (
            jnp.dot(h, wg1_ref[...], preferred_element_type=jnp.float32)
            + bg1_ref[...]
        )

    @pl.when((i >= 1) & (i <= nblk))
    def _():
        t = jnp.dot(adj_ref[...], h1_ref[...], preferred_element_type=jnp.float32)
        t = jnp.maximum(t, 0.0)
        h2_ref[pl.ds((i - 1) * bm, bm), :] = (
            jnp.dot(t, wg2_ref[...], preferred_element_type=jnp.float32)
            + bg2_ref[...]
        )

    @pl.when(i > nblk)
    def _():
        t = jnp.dot(
            adj_ref[...], h2_ref[:n, :], preferred_element_type=jnp.float32
        )
        out_ref[...] = jnp.maximum(t, 0.0)


def kernel(x, adj, W_fc, b_fc, W_g1, b_g1, W_g2, b_g2):
    n, in_ft = x.shape
    h1w = W_g1.shape[1]
    outw = W_g2.shape[1]
    b_fc2 = b_fc.reshape(1, -1)
    b_g12 = b_g1.reshape(1, -1)
    b_g22 = b_g2.reshape(1, -1)

    bm = min(512, ((n + 7) // 8) * 8)
    nblk = pl.cdiv(n, bm)

    full = lambda shape: pl.BlockSpec(shape, lambda i: (0, 0))

    out = pl.pallas_call(
        functools.partial(_fused_kernel, n=n, nblk=nblk, bm=bm),
        grid=(2 * nblk + 1,),
        in_specs=[
            full((n, in_ft)),
            pl.BlockSpec((bm, n), lambda i: ((jnp.maximum(i, 1) - 1) % nblk, 0)),
            full(W_fc.shape),
            full(b_fc2.shape),
            full(W_g1.shape),
            full(b_g12.shape),
            full(W_g2.shape),
            full(b_g22.shape),
        ],
        out_specs=pl.BlockSpec(
            (bm, outw), lambda i: (jnp.maximum(i - (nblk + 1), 0), 0)
        ),
        out_shape=jax.ShapeDtypeStruct((n, outw), jnp.float32),
        scratch_shapes=[
            pltpu.VMEM((n, h1w), jnp.float32),
            pltpu.VMEM((nblk * bm, outw), jnp.float32),
        ],
        compiler_params=pltpu.CompilerParams(
            vmem_limit_bytes=64 * 1024 * 1024,
        ),
    )(x, adj, W_fc, b_fc2, W_g1, b_g12, W_g2, b_g22)
    return out
